# baseline (speedup 1.0000x reference)
import functools
import numpy as np
import jax
import jax.numpy as jnp
from jax.experimental import pallas as pl
from jax.experimental.pallas import tpu as pltpu

_D = 32
_H = 4
_HD = _D // _H
_N = 5
_PATCH = 16
_CHANS = 3
_IMG = 32
_KFLAT = _CHANS * _IMG * _IMG
_NUM_CLASSES = 10
_EPS = 1e-6
_GELU_C = float(np.sqrt(2.0 / np.pi))
_PACK = 4
_DP = _PACK * _D


def _gelu_tanh(v):
    return 0.5 * v * (1.0 + jnp.tanh(_GELU_C * (v + 0.044715 * v * v * v)))


def _vit_kernel(x0_ref, x1_ref, x2_ref, x3_ref, wbig_ref, vec32_ref,
                f1b_ref, wq_ref, bq_ref, wk_ref, bk_ref, wv_ref, bv_ref,
                red_ref, b32_ref, projw_ref, fc1w_ref, fc2w_ref, headw_ref,
                hb_ref, o_ref, *, g, depth):
    xrefs = [x0_ref, x1_ref, x2_ref, x3_ref]
    def vrow(r):
        return vec32_ref[pl.ds(r, 1), :]

    def norm(v, r):
        vb = v.astype(jnp.bfloat16)
        sq = (v * v).astype(jnp.bfloat16)
        mu = jnp.dot(vb, b32_ref[...], preferred_element_type=jnp.float32)
        ms = jnp.dot(sq, b32_ref[...], preferred_element_type=jnp.float32)
        var = ms - mu * mu
        return (v - mu) * jax.lax.rsqrt(var + _EPS) * vrow(r) + vrow(r + 1)

    eparts = [jnp.dot(xrefs[kk][...].reshape(g, _KFLAT).astype(jnp.bfloat16),
                      wbig_ref[...], preferred_element_type=jnp.float32)
              for kk in range(_PACK)]
    e4 = jnp.concatenate(eparts, axis=1)

    toks = [jnp.broadcast_to(vrow(0), (g, _DP))]
    for p in range(_N - 1):
        slab = jnp.concatenate(
            [e4[:, kk * _DP + p * _D: kk * _DP + (p + 1) * _D]
             for kk in range(_PACK)], axis=1)
        toks.append(slab + vrow(p + 1))
    x = jnp.concatenate(toks, axis=0)

    for l in range(depth):
        base = _N + 6 * l
        hb = norm(x, base).astype(jnp.bfloat16)
        q = jnp.dot(hb, wq_ref[l], preferred_element_type=jnp.float32) + bq_ref[l]
        k = jnp.dot(hb, wk_ref[l], preferred_element_type=jnp.float32) + bk_ref[l]
        v = jnp.dot(hb, wv_ref[l], preferred_element_type=jnp.float32) + bv_ref[l]

        ctxs = []
        for a in range(_N):
            qa = q[a * g:(a + 1) * g]
            logits = [jnp.dot((qa * k[b * g:(b + 1) * g]).astype(jnp.bfloat16),
                              red_ref[...], preferred_element_type=jnp.float32)
                      for b in range(_N)]
            m = logits[0]
            for b in range(1, _N):
                m = jnp.maximum(m, logits[b])
            exps = [jnp.exp(lg - m) for lg in logits]
            z = exps[0]
            for b in range(1, _N):
                z = z + exps[b]
            num = exps[0] * v[0:g]
            for b in range(1, _N):
                num = num + exps[b] * v[b * g:(b + 1) * g]
            ctxs.append(num / z)
        ctx = jnp.concatenate(ctxs, axis=0).astype(jnp.bfloat16)
        x = x + jnp.dot(ctx, projw_ref[l],
                        preferred_element_type=jnp.float32) + vrow(base + 2)

        hm = norm(x, base + 3).astype(jnp.bfloat16)
        hm = jnp.dot(hm, fc1w_ref[l],
                     preferred_element_type=jnp.float32) + f1b_ref[pl.ds(l, 1), :]
        hm = _gelu_tanh(hm).astype(jnp.bfloat16)
        x = x + jnp.dot(hm, fc2w_ref[l],
                        preferred_element_type=jnp.float32) + vrow(base + 5)

    cls = norm(x[0:g], _N + 6 * depth).astype(jnp.bfloat16)
    logits4 = jnp.dot(cls, headw_ref[...],
                      preferred_element_type=jnp.float32) + hb_ref[...]
    for kk in range(_PACK):
        o_ref[kk * g:(kk + 1) * g, :] = \
            logits4[:, kk * _NUM_CLASSES:(kk + 1) * _NUM_CLASSES]


def _kron4(w):
    i4 = jnp.eye(_PACK, dtype=w.dtype)
    a, b = w.shape[-2], w.shape[-1]
    lead = w.shape[:-2]
    out = jnp.einsum('...ab,ij->...iajb', w, i4)
    return out.reshape(*lead, _PACK * a, _PACK * b)


def kernel(x, patch_w, vec32, vec128, wqkv, bqkv, attn_mask, proj_w,
           fc1_w, fc2_w, head_w):
    del attn_mask
    B = x.shape[0]
    L = wqkv.shape[0]
    scale = float(_HD) ** -0.5

    eye2 = jnp.eye(2, dtype=patch_w.dtype)
    w4 = patch_w.reshape(_CHANS, _PATCH, _PATCH, _D)
    wbig = jnp.einsum('cabe,hH,wV->chawbHVe', w4, eye2, eye2)
    wbig = wbig.reshape(_KFLAT, (_N - 1) * _D).astype(jnp.bfloat16)

    vec32_4 = jnp.tile(vec32, (1, _PACK))
    f1b4 = jnp.tile(vec128[:L], (1, _PACK))

    wq_all = wqkv.reshape(L, _D, _H, 3, _HD).transpose(0, 1, 3, 2, 4)
    wq_all = wq_all.reshape(L, _D, 3 * _D)
    bq_all = bqkv.reshape(L, 1, _H, 3, _HD).transpose(0, 1, 3, 2, 4)
    bq_all = bq_all.reshape(L, 1, 3 * _D)

    def qkv_part(s, mult):
        w = _kron4(wq_all[:, :, s * _D:(s + 1) * _D] * mult).astype(jnp.bfloat16)
        b = jnp.tile(bq_all[:, :, s * _D:(s + 1) * _D] * mult, (1, 1, _PACK))
        return w, b

    wq4, bq4 = qkv_part(0, scale)
    wk4, bk4 = qkv_part(1, 1.0)
    wv4, bv4 = qkv_part(2, 1.0)

    proj4 = _kron4(proj_w).astype(jnp.bfloat16)
    fc14 = _kron4(fc1_w).astype(jnp.bfloat16)
    fc24 = _kron4(fc2_w).astype(jnp.bfloat16)
    head4 = _kron4(head_w[:, :_NUM_CLASSES]).astype(jnp.bfloat16)
    hb4 = jnp.tile(vec128[L:L + 1, :_NUM_CLASSES], (1, _PACK))

    red = jnp.asarray(np.kron(np.eye(_H * _PACK), np.ones((_HD, _HD))),
                      dtype=jnp.bfloat16)
    b32 = jnp.asarray(np.kron(np.eye(_PACK), np.ones((_D, _D)) / _D),
                      dtype=jnp.bfloat16)

    x12 = x.reshape(B, _CHANS * 4, 8, _IMG)

    tb = 512
    while B % tb or tb % _PACK:
        tb //= 2
    g = tb // _PACK
    grid = (B // tb,)

    weights = [wbig, vec32_4, f1b4, wq4, bq4, wk4, bk4, wv4, bv4, red, b32,
               proj4, fc14, fc24, head4, hb4]

    def fixed(a):
        nd = a.ndim
        return pl.BlockSpec(a.shape, lambda i, _nd=nd: (0,) * _nd)

    xspecs = [pl.BlockSpec((g, _CHANS * 4, 8, _IMG),
                           lambda i, _k=kk: (i * _PACK + _k, 0, 0, 0))
              for kk in range(_PACK)]

    kern = functools.partial(_vit_kernel, g=g, depth=L)
    out = pl.pallas_call(
        kern,
        grid=grid,
        out_shape=jax.ShapeDtypeStruct((B, _NUM_CLASSES), jnp.float32),
        in_specs=xspecs + [fixed(a) for a in weights],
        out_specs=pl.BlockSpec((tb, _NUM_CLASSES), lambda i: (i, 0)),
        compiler_params=pltpu.CompilerParams(
            dimension_semantics=("parallel",)),
    )(x12, x12, x12, x12, *weights)
    return out


# speedup vs baseline: 1.3257x; 1.3257x over previous
import functools
import numpy as np
import jax
import jax.numpy as jnp
from jax.experimental import pallas as pl
from jax.experimental.pallas import tpu as pltpu

_D = 32
_H = 4
_HD = _D // _H
_N = 5
_PATCH = 16
_CHANS = 3
_IMG = 32
_KFLAT = _CHANS * _IMG * _IMG
_NUM_CLASSES = 10
_EPS = 1e-6
_GELU_C = float(np.sqrt(2.0 / np.pi))
_PACK = 4
_DP = _PACK * _D


def _gelu_tanh(v):
    return 0.5 * v * (1.0 + jnp.tanh(_GELU_C * (v + 0.044715 * v * v * v)))


def _vit_kernel(x0_ref, x1_ref, x2_ref, x3_ref, wbig_ref, vec32_ref,
                f1b_ref, wq_ref, bq_ref, wk_ref, bk_ref, wv_ref, bv_ref,
                red_ref, b32_ref, projw_ref, fc1w_ref, fc2w_ref, headw_ref,
                hb_ref, o_ref, *, g, depth):
    xrefs = [x0_ref, x1_ref, x2_ref, x3_ref]
    def vrow(r):
        return vec32_ref[pl.ds(r, 1), :]

    def norm(v, r):
        vb = v.astype(jnp.bfloat16)
        sq = (v * v).astype(jnp.bfloat16)
        mu = jnp.dot(vb, b32_ref[...], preferred_element_type=jnp.float32)
        ms = jnp.dot(sq, b32_ref[...], preferred_element_type=jnp.float32)
        var = ms - mu * mu
        return (v - mu) * jax.lax.rsqrt(var + _EPS) * vrow(r) + vrow(r + 1)

    eparts = [jnp.dot(xrefs[kk][...].astype(jnp.bfloat16), wbig_ref[...],
                      preferred_element_type=jnp.float32)
              for kk in range(_PACK)]
    e4 = jnp.concatenate(eparts, axis=1)

    toks = [jnp.broadcast_to(vrow(0), (g, _DP))]
    for p in range(_N - 1):
        slab = jnp.concatenate(
            [e4[:, kk * _DP + p * _D: kk * _DP + (p + 1) * _D]
             for kk in range(_PACK)], axis=1)
        toks.append(slab + vrow(p + 1))
    x = jnp.concatenate(toks, axis=0)

    for l in range(depth):
        base = _N + 6 * l
        hb = norm(x, base).astype(jnp.bfloat16)
        q = jnp.dot(hb, wq_ref[l], preferred_element_type=jnp.float32) + bq_ref[l]
        k = jnp.dot(hb, wk_ref[l], preferred_element_type=jnp.float32) + bk_ref[l]
        v = jnp.dot(hb, wv_ref[l], preferred_element_type=jnp.float32) + bv_ref[l]

        ctxs = []
        for a in range(_N):
            qa = q[a * g:(a + 1) * g]
            logits = [jnp.dot((qa * k[b * g:(b + 1) * g]).astype(jnp.bfloat16),
                              red_ref[...], preferred_element_type=jnp.float32)
                      for b in range(_N)]
            m = logits[0]
            for b in range(1, _N):
                m = jnp.maximum(m, logits[b])
            exps = [jnp.exp(lg - m) for lg in logits]
            z = exps[0]
            for b in range(1, _N):
                z = z + exps[b]
            num = exps[0] * v[0:g]
            for b in range(1, _N):
                num = num + exps[b] * v[b * g:(b + 1) * g]
            ctxs.append(num / z)
        ctx = jnp.concatenate(ctxs, axis=0).astype(jnp.bfloat16)
        x = x + jnp.dot(ctx, projw_ref[l],
                        preferred_element_type=jnp.float32) + vrow(base + 2)

        hm = norm(x, base + 3).astype(jnp.bfloat16)
        hm = jnp.dot(hm, fc1w_ref[l],
                     preferred_element_type=jnp.float32) + f1b_ref[pl.ds(l, 1), :]
        hm = _gelu_tanh(hm).astype(jnp.bfloat16)
        x = x + jnp.dot(hm, fc2w_ref[l],
                        preferred_element_type=jnp.float32) + vrow(base + 5)

    cls = norm(x[0:g], _N + 6 * depth).astype(jnp.bfloat16)
    logits4 = jnp.dot(cls, headw_ref[...],
                      preferred_element_type=jnp.float32) + hb_ref[...]
    for kk in range(_PACK):
        o_ref[kk * g:(kk + 1) * g, :] = \
            logits4[:, kk * _NUM_CLASSES:(kk + 1) * _NUM_CLASSES]


def _kron4(w):
    i4 = jnp.eye(_PACK, dtype=w.dtype)
    a, b = w.shape[-2], w.shape[-1]
    lead = w.shape[:-2]
    out = jnp.einsum('...ab,ij->...iajb', w, i4)
    return out.reshape(*lead, _PACK * a, _PACK * b)


def kernel(x, patch_w, vec32, vec128, wqkv, bqkv, attn_mask, proj_w,
           fc1_w, fc2_w, head_w):
    del attn_mask
    B = x.shape[0]
    L = wqkv.shape[0]
    scale = float(_HD) ** -0.5

    eye2 = jnp.eye(2, dtype=patch_w.dtype)
    w4 = patch_w.reshape(_CHANS, _PATCH, _PATCH, _D)
    wbig = jnp.einsum('cabe,hH,wV->chawbHVe', w4, eye2, eye2)
    wbig = wbig.reshape(_KFLAT, (_N - 1) * _D).astype(jnp.bfloat16)

    vec32_4 = jnp.tile(vec32, (1, _PACK))
    f1b4 = jnp.tile(vec128[:L], (1, _PACK))

    wq_all = wqkv.reshape(L, _D, _H, 3, _HD).transpose(0, 1, 3, 2, 4)
    wq_all = wq_all.reshape(L, _D, 3 * _D)
    bq_all = bqkv.reshape(L, 1, _H, 3, _HD).transpose(0, 1, 3, 2, 4)
    bq_all = bq_all.reshape(L, 1, 3 * _D)

    def qkv_part(s, mult):
        w = _kron4(wq_all[:, :, s * _D:(s + 1) * _D] * mult).astype(jnp.bfloat16)
        b = jnp.tile(bq_all[:, :, s * _D:(s + 1) * _D] * mult, (1, 1, _PACK))
        return w, b

    wq4, bq4 = qkv_part(0, scale)
    wk4, bk4 = qkv_part(1, 1.0)
    wv4, bv4 = qkv_part(2, 1.0)

    proj4 = _kron4(proj_w).astype(jnp.bfloat16)
    fc14 = _kron4(fc1_w).astype(jnp.bfloat16)
    fc24 = _kron4(fc2_w).astype(jnp.bfloat16)
    head4 = _kron4(head_w[:, :_NUM_CLASSES]).astype(jnp.bfloat16)
    hb4 = jnp.tile(vec128[L:L + 1, :_NUM_CLASSES], (1, _PACK))

    red = jnp.asarray(np.kron(np.eye(_H * _PACK), np.ones((_HD, _HD))),
                      dtype=jnp.bfloat16)
    b32 = jnp.asarray(np.kron(np.eye(_PACK), np.ones((_D, _D)) / _D),
                      dtype=jnp.bfloat16)

    x2 = x.reshape(B, _KFLAT)

    tb = 512
    while B % tb or tb % _PACK:
        tb //= 2
    g = tb // _PACK
    grid = (B // tb,)

    weights = [wbig, vec32_4, f1b4, wq4, bq4, wk4, bk4, wv4, bv4, red, b32,
               proj4, fc14, fc24, head4, hb4]

    def fixed(a):
        nd = a.ndim
        return pl.BlockSpec(a.shape, lambda i, _nd=nd: (0,) * _nd)

    xspecs = [pl.BlockSpec((g, _KFLAT), lambda i, _k=kk: (i * _PACK + _k, 0))
              for kk in range(_PACK)]

    kern = functools.partial(_vit_kernel, g=g, depth=L)
    out = pl.pallas_call(
        kern,
        grid=grid,
        out_shape=jax.ShapeDtypeStruct((B, _NUM_CLASSES), jnp.float32),
        in_specs=xspecs + [fixed(a) for a in weights],
        out_specs=pl.BlockSpec((tb, _NUM_CLASSES), lambda i: (i, 0)),
        compiler_params=pltpu.CompilerParams(
            dimension_semantics=("parallel",)),
    )(x2, x2, x2, x2, *weights)
    return out
